# Initial kernel scaffold; baseline (speedup 1.0000x reference)
import jax
import jax.numpy as jnp
from jax.experimental import pallas as pl


def kernel(hidden_states, cu_seqlens):
    raise NotImplementedError("write your pallas kernel here")


# speedup vs baseline: 2.4734x; 2.4734x over previous
import functools

import jax
import jax.numpy as jnp
from jax import lax
from jax.experimental import pallas as pl
from jax.experimental.pallas import tpu as pltpu
from jax.experimental.pallas import tpu_sc as plsc

NC = 2
NS = 16
NW = NC * NS
L = 16

BIG = 0x7FFFFFFF


def _body(T, N, D, rows_per_tile, n_chunks, hid_hbm, cu_hbm, out_hbm,
          cu_v, ocu_v, buf_v, sem):
    wid = lax.axis_index("s") * NC + lax.axis_index("c")
    base = wid * rows_per_tile

    pltpu.sync_copy(cu_hbm, cu_v)

    for h in range(2):
        j = jnp.int32(h * L) + lax.iota(jnp.int32, L)
        idx = jnp.maximum(jnp.int32(N) - j, 0)
        val = plsc.load_gather(cu_v, [idx])
        ocu = jnp.where(j <= N, jnp.int32(T) - val, jnp.int32(BIG))
        ocu_v[pl.ds(h * L, L)] = ocu

    def chunk(g, carry):
        p = base + g * L + lax.iota(jnp.int32, L)
        lo = jnp.zeros((L,), jnp.int32)
        for step in (16, 8, 4, 2, 1):
            cand = lo + jnp.int32(step)
            v = plsc.load_gather(ocu_v, [cand])
            lo = jnp.where(v <= p, cand, lo)
        src = (plsc.load_gather(cu_v, [jnp.int32(N - 1) - lo])
               + plsc.load_gather(cu_v, [jnp.int32(N) - lo])
               + p - jnp.int32(T))
        pltpu.async_copy(hid_hbm.at[src], buf_v, sem).wait()
        pltpu.sync_copy(buf_v, out_hbm.at[pl.ds(base + g * L, L)])
        return carry

    lax.fori_loop(0, n_chunks, chunk, jnp.int32(0))


def kernel(hidden_states, cu_seqlens):
    T, D = hidden_states.shape
    N = cu_seqlens.shape[0] - 1
    rows_per_tile = T // NW
    n_chunks = rows_per_tile // L

    cu_pad = jnp.concatenate(
        [cu_seqlens.astype(jnp.int32),
         jnp.full((32 - (N + 1),), BIG, dtype=jnp.int32)])

    mesh = plsc.VectorSubcoreMesh(core_axis_name="c", subcore_axis_name="s")
    body = functools.partial(_body, T, N, D, rows_per_tile, n_chunks)
    f = pl.kernel(
        body,
        out_type=jax.ShapeDtypeStruct((T, D), jnp.float32),
        mesh=mesh,
        compiler_params=pltpu.CompilerParams(needs_layout_passes=False),
        scratch_types=[
            pltpu.VMEM((32,), jnp.int32),
            pltpu.VMEM((32,), jnp.int32),
            pltpu.VMEM((L, D), jnp.float32),
            pltpu.SemaphoreType.DMA,
        ],
    )
    return f(hidden_states, cu_pad)


# speedup vs baseline: 2.8189x; 1.1397x over previous
import functools

import jax
import jax.numpy as jnp
from jax import lax
from jax.experimental import pallas as pl
from jax.experimental.pallas import tpu as pltpu
from jax.experimental.pallas import tpu_sc as plsc

NC = 2
NS = 16
NW = NC * NS
L = 16

BIG = 0x7FFFFFFF


def _body(T, N, D, rows_per_tile, n_chunks, hid_hbm, cu_hbm, out_hbm,
          cu_v, ocu_v, buf0, buf1, gsem0, gsem1, ssem0, ssem1):
    wid = lax.axis_index("s") * NC + lax.axis_index("c")
    base = wid * rows_per_tile
    bufs = (buf0, buf1)
    gsems = (gsem0, gsem1)
    ssems = (ssem0, ssem1)

    pltpu.sync_copy(cu_hbm, cu_v)

    for h in range(2):
        j = jnp.int32(h * L) + lax.iota(jnp.int32, L)
        idx = jnp.maximum(jnp.int32(N) - j, 0)
        val = plsc.load_gather(cu_v, [idx])
        ocu = jnp.where(j <= N, jnp.int32(T) - val, jnp.int32(BIG))
        ocu_v[pl.ds(h * L, L)] = ocu

    def src_of(g):
        p = base + jnp.int32(g * L) + lax.iota(jnp.int32, L)
        lo = jnp.zeros((L,), jnp.int32)
        for step in (16, 8, 4, 2, 1):
            cand = lo + jnp.int32(step)
            v = plsc.load_gather(ocu_v, [cand])
            lo = jnp.where(v <= p, cand, lo)
        return (plsc.load_gather(cu_v, [jnp.int32(N - 1) - lo])
                + plsc.load_gather(cu_v, [jnp.int32(N) - lo])
                + p - jnp.int32(T))

    gd = [None] * n_chunks
    sd = [None] * n_chunks
    for g in range(n_chunks):
        b = g % 2
        if g >= 2:
            sd[g - 2].wait()
        gd[g] = pltpu.async_copy(hid_hbm.at[src_of(g)], bufs[b], gsems[b])
        if g >= 1:
            gd[g - 1].wait()
            sd[g - 1] = pltpu.async_copy(
                bufs[1 - b], out_hbm.at[pl.ds(base + (g - 1) * L, L)],
                ssems[1 - b])
    g = n_chunks - 1
    gd[g].wait()
    sd[g] = pltpu.async_copy(
        bufs[g % 2], out_hbm.at[pl.ds(base + g * L, L)], ssems[g % 2])
    sd[g - 1].wait()
    sd[g].wait()


def kernel(hidden_states, cu_seqlens):
    T, D = hidden_states.shape
    N = cu_seqlens.shape[0] - 1
    rows_per_tile = T // NW
    n_chunks = rows_per_tile // L

    cu_pad = jnp.concatenate(
        [cu_seqlens.astype(jnp.int32),
         jnp.full((32 - (N + 1),), BIG, dtype=jnp.int32)])

    mesh = plsc.VectorSubcoreMesh(core_axis_name="c", subcore_axis_name="s")
    body = functools.partial(_body, T, N, D, rows_per_tile, n_chunks)
    f = pl.kernel(
        body,
        out_type=jax.ShapeDtypeStruct((T, D), jnp.float32),
        mesh=mesh,
        compiler_params=pltpu.CompilerParams(needs_layout_passes=False),
        scratch_types=[
            pltpu.VMEM((32,), jnp.int32),
            pltpu.VMEM((32,), jnp.int32),
            pltpu.VMEM((L, D), jnp.float32),
            pltpu.VMEM((L, D), jnp.float32),
            pltpu.SemaphoreType.DMA,
            pltpu.SemaphoreType.DMA,
            pltpu.SemaphoreType.DMA,
            pltpu.SemaphoreType.DMA,
        ],
    )
    return f(hidden_states, cu_pad)
